# baseline (speedup 1.0000x reference)
import functools

import jax
import jax.numpy as jnp
from jax import lax
from jax.experimental import pallas as pl
from jax.experimental.pallas import tpu as pltpu
from jax.experimental.pallas import tpu_sc as plsc

_L = 16
_IDX = 480
_CHUNK = 1600
_W = _CHUNK * 3
_NIDX = _W // _IDX
_NC, _NS = 2, 16
_NW = _NC * _NS


def _rsqrt(x):
    i = plsc.bitcast(x, jnp.int32)
    y = plsc.bitcast(jnp.int32(0x5F3759DF) - lax.shift_right_logical(i, 1),
                     jnp.float32)
    y = y * (1.5 - 0.5 * x * y * y)
    y = y * (1.5 - 0.5 * x * y * y)
    y = y * (1.5 - 0.5 * x * y * y)
    return y


@functools.partial(jax.jit, static_argnums=(0, 1, 2))
def _sc_edges(n, n_pad, e, dipf, ii, jj, vx, vy, vz):
    n_chunks = e // _CHUNK
    max_my = -(-n_chunks // _NW)
    n_pairs = -(-max_my // 2)
    words_per_sub = 3 * n_pad // _NS
    n_wb = words_per_sub // _W
    mesh = plsc.VectorSubcoreMesh(core_axis_name="c", subcore_axis_name="s")

    @functools.partial(
        pl.kernel,
        out_type=jax.ShapeDtypeStruct((_NC, 3 * n_pad), jnp.float32),
        mesh=mesh,
        compiler_params=pltpu.CompilerParams(
            use_tc_tiling_on_sc=False, needs_layout_passes=False),
        scratch_types=[
            pltpu.VMEM((_W,), jnp.float32),
            pltpu.VMEM((_W,), jnp.float32),
            pltpu.VMEM((_CHUNK,), jnp.int32),
            pltpu.VMEM((_CHUNK,), jnp.int32),
            pltpu.VMEM((_W,), jnp.int32),
            pltpu.VMEM((_W,), jnp.int32),
            pltpu.VMEM((_W,), jnp.int32),
            pltpu.VMEM((_W,), jnp.int32),
            pltpu.VMEM((_W,), jnp.float32),
            pltpu.VMEM((_W,), jnp.float32),
            pltpu.VMEM((_W,), jnp.float32),
            pltpu.VMEM((_W,), jnp.float32),
            pltpu.VMEM((_W,), jnp.float32),
            pltpu.VMEM((_W,), jnp.float32),
            pltpu.VMEM((_W,), jnp.float32),
            pltpu.VMEM((_W,), jnp.float32),
            pltpu.VMEM_SHARED((3 * n_pad,), jnp.float32),
            pltpu.VMEM_SHARED((3 * n_pad,), jnp.float32),
            pltpu.SemaphoreType.DMA,
            pltpu.SemaphoreType.DMA,
            pltpu.SemaphoreType.DMA,
            pltpu.SemaphoreType.DMA,
            pltpu.SemaphoreType.DMA,
        ],
    )
    def sc_kernel(dip_hbm, ii_hbm, jj_hbm, vx_hbm, vy_hbm, vz_hbm, out_hbm,
                  vecb0, vecb1, ib, jb, eib0, eib1, ejb0, ejb1,
                  dib0, dib1, djb0, djb1,
                  cib0, cib1, cjb0, cjb1, acc, dipm,
                  sem_in, sem_g0, sem_g1, sem_s0, sem_s1):
        c = lax.axis_index("c")
        s = lax.axis_index("s")
        wid = s * _NC + c
        zf = jnp.zeros((_L,), jnp.float32)
        vecbs = (vecb0, vecb1)
        eibs, ejbs = (eib0, eib1), (ejb0, ejb1)
        dibs, djbs = (dib0, dib1), (djb0, djb1)
        cibs, cjbs = (cib0, cib1), (cjb0, cjb1)
        sems = (sem_s0, sem_s1)
        gsems = (sem_g0, sem_g1)
        n_my = (n_chunks - wid + _NW - 1) // _NW

        def zbuf_body(v, carry):
            cib0[pl.ds(v * _L, _L)] = zf
            return carry
        lax.fori_loop(0, _W // _L, zbuf_body, 0)
        w0_sub = s * words_per_sub

        def zacc_body(p, carry):
            pltpu.sync_copy(cib0, acc.at[pl.ds(w0_sub + p * _W, _W)])
            return carry
        lax.fori_loop(0, n_wb, zacc_body, 0)

        def stage_body(p, carry):
            off = w0_sub + p * _W
            pltpu.sync_copy(dip_hbm.at[pl.ds(off, _W)], cib0)
            pltpu.sync_copy(cib0, dipm.at[pl.ds(off, _W)])
            return carry
        lax.fori_loop(0, n_wb, stage_body, 0)
        plsc.subcore_barrier()

        def scatter_descs(h, issue):
            ds_list = []
            for k in range(_NIDX):
                sl = pl.ds(k * _IDX, _IDX)
                fn = pltpu.async_copy if issue else pltpu.make_async_copy
                kw = {"add": True} if issue else {}
                ds_list.append(fn(
                    cibs[h].at[sl], acc.at[eibs[h].at[sl]], sems[h], **kw))
                ds_list.append(fn(
                    cjbs[h].at[sl], acc.at[ejbs[h].at[sl]], sems[h], **kw))
            return ds_list

        def chunk_front(t, h):
            ch = wid + t * _NW
            base = ch * _CHUNK
            vecb = vecbs[h]
            cps = [
                pltpu.async_copy(
                    vx_hbm.at[pl.ds(base, _CHUNK)],
                    vecb.at[pl.ds(0, _CHUNK)], sem_in),
                pltpu.async_copy(
                    vy_hbm.at[pl.ds(base, _CHUNK)],
                    vecb.at[pl.ds(_CHUNK, _CHUNK)], sem_in),
                pltpu.async_copy(
                    vz_hbm.at[pl.ds(base, _CHUNK)],
                    vecb.at[pl.ds(2 * _CHUNK, _CHUNK)], sem_in),
                pltpu.async_copy(
                    ii_hbm.at[pl.ds(base, _CHUNK)], ib, sem_in),
                pltpu.async_copy(
                    jj_hbm.at[pl.ds(base, _CHUNK)], jb, sem_in),
            ]
            for cp in cps:
                cp.wait()
            eib, ejb = eibs[h], ejbs[h]

            def exp_body(v, carry2):
                sl = pl.ds(v * _L, _L)
                iv = ib[sl]
                jv = jb[sl]
                eib[sl] = iv
                eib[pl.ds(_CHUNK + v * _L, _L)] = iv + n_pad
                eib[pl.ds(2 * _CHUNK + v * _L, _L)] = iv + 2 * n_pad
                ejb[sl] = jv
                ejb[pl.ds(_CHUNK + v * _L, _L)] = jv + n_pad
                ejb[pl.ds(2 * _CHUNK + v * _L, _L)] = jv + 2 * n_pad
                return carry2
            lax.fori_loop(0, _CHUNK // _L, exp_body, 0)

            for k in range(_NIDX):
                sl = pl.ds(k * _IDX, _IDX)
                pltpu.async_copy(dipm.at[eib.at[sl]], dibs[h].at[sl], gsems[h])
                pltpu.async_copy(dipm.at[ejb.at[sl]], djbs[h].at[sl], gsems[h])

        def chunk_back(t, h):
            vecb, dib, djb = vecbs[h], dibs[h], djbs[h]
            eib, ejb = eibs[h], ejbs[h]
            cib, cjb = cibs[h], cjbs[h]
            for k in range(_NIDX):
                sl = pl.ds(k * _IDX, _IDX)
                pltpu.make_async_copy(
                    dipm.at[eib.at[sl]], dib.at[sl], gsems[h]).wait()
                pltpu.make_async_copy(
                    dipm.at[ejb.at[sl]], djb.at[sl], gsems[h]).wait()

            def vec_body(v, carry2):
                s0 = pl.ds(v * _L, _L)
                s1 = pl.ds(_CHUNK + v * _L, _L)
                s2 = pl.ds(2 * _CHUNK + v * _L, _L)
                rx = vecb[s0]
                ry = vecb[s1]
                rz = vecb[s2]
                dix = dib[s0]
                diy = dib[s1]
                diz = dib[s2]
                djx = djb[s0]
                djy = djb[s1]
                djz = djb[s2]
                d2 = rx * rx + ry * ry + rz * rz
                y = _rsqrt(d2)
                y2 = y * y
                a3 = 0.5 * (y2 * y)
                a5 = 3.0 * (a3 * y2)
                sj = a5 * (rx * djx + ry * djy + rz * djz)
                cib[s0] = sj * rx - a3 * djx
                cib[s1] = sj * ry - a3 * djy
                cib[s2] = sj * rz - a3 * djz
                si = a5 * (rx * dix + ry * diy + rz * diz)
                cjb[s0] = si * rx - a3 * dix
                cjb[s1] = si * ry - a3 * diy
                cjb[s2] = si * rz - a3 * diz
                return carry2
            lax.fori_loop(0, _CHUNK // _L, vec_body, 0)

            scatter_descs(h, issue=True)

        def pair_body(t2, carry):
            for h in (0, 1):
                t = 2 * t2 + h

                @pl.when((t - 2 >= 0) & (t - 2 < n_my))
                def _drain():
                    for d in scatter_descs(h, issue=False):
                        d.wait()

                @pl.when(t < n_my)
                def _front():
                    chunk_front(t, h)
            for h in (0, 1):
                t = 2 * t2 + h

                @pl.when(t < n_my)
                def _back():
                    chunk_back(t, h)
            return carry
        lax.fori_loop(0, n_pairs, pair_body, 0)

        for h in (0, 1):
            last_t = n_my - 1 - ((n_my - 1 - h) % 2)

            @pl.when((last_t >= 0) & (last_t + 2 >= (2 * n_pairs)))
            def _tail_drain():
                for d in scatter_descs(h, issue=False):
                    d.wait()
        plsc.subcore_barrier()

        def wb_body(p, carry):
            off = w0_sub + p * _W
            pltpu.sync_copy(acc.at[pl.ds(off, _W)], cib0)
            pltpu.sync_copy(cib0, out_hbm.at[c].at[pl.ds(off, _W)])
            return carry
        lax.fori_loop(0, n_wb, wb_body, 0)

    return sc_kernel(dipf, ii, jj, vx, vy, vz)


def _combine(partials):
    def body(i_ref, o_ref):
        o_ref[...] = i_ref[0] + i_ref[1]
    return pl.pallas_call(
        body,
        out_shape=jax.ShapeDtypeStruct(partials.shape[1:], jnp.float32),
    )(partials)


def kernel(dipoles, cell, positions, neighbor_indices, neighbor_vectors):
    n = dipoles.shape[0]
    e = neighbor_vectors.shape[0]
    assert e % _CHUNK == 0
    blk = _NS * _CHUNK
    n_pad = -(-n // blk) * blk
    idx_t = neighbor_indices.T
    ii = idx_t[0]
    jj = idx_t[1]
    vx = neighbor_vectors[:, 0]
    vy = neighbor_vectors[:, 1]
    vz = neighbor_vectors[:, 2]
    dip_t = dipoles.T
    dipf = jnp.zeros((3, n_pad), dipoles.dtype).at[:, :n].set(dip_t).reshape(-1)
    partials = _sc_edges(n, n_pad, e, dipf, ii, jj, vx, vy, vz)
    summed = _combine(partials.reshape(_NC, 3 * n_pad // 128, 128))
    return summed.reshape(3, n_pad)[:, :n].T


# speedup vs baseline: 1.0896x; 1.0896x over previous
import functools

import jax
import jax.numpy as jnp
from jax import lax
from jax.experimental import pallas as pl
from jax.experimental.pallas import tpu as pltpu
from jax.experimental.pallas import tpu_sc as plsc

_L = 16
_IDX = 512
_CHUNK = 2048
_W = _CHUNK * 3
_NIDX = _W // _IDX
_NC, _NS = 2, 16
_NW = _NC * _NS


def _rsqrt(x):
    i = plsc.bitcast(x, jnp.int32)
    y = plsc.bitcast(jnp.int32(0x5F3759DF) - lax.shift_right_logical(i, 1),
                     jnp.float32)
    y = y * (1.5 - 0.5 * x * y * y)
    y = y * (1.5 - 0.5 * x * y * y)
    y = y * (1.5 - 0.5 * x * y * y)
    return y


@functools.partial(jax.jit, static_argnums=(0, 1, 2))
def _sc_edges(n, n_pad, e, dipf, ii, jj, vx, vy, vz):
    n_chunks = e // _CHUNK
    max_my = -(-n_chunks // _NW)
    n_pairs = -(-max_my // 2)
    words_per_sub = 3 * n_pad // _NS
    n_wb = words_per_sub // _W
    mesh = plsc.VectorSubcoreMesh(core_axis_name="c", subcore_axis_name="s")

    @functools.partial(
        pl.kernel,
        out_type=jax.ShapeDtypeStruct((_NC, 3 * n_pad), jnp.float32),
        mesh=mesh,
        compiler_params=pltpu.CompilerParams(
            use_tc_tiling_on_sc=False, needs_layout_passes=False),
        scratch_types=[
            pltpu.VMEM((_W,), jnp.float32),
            pltpu.VMEM((_CHUNK,), jnp.int32),
            pltpu.VMEM((_CHUNK,), jnp.int32),
            pltpu.VMEM((_W,), jnp.int32),
            pltpu.VMEM((_W,), jnp.int32),
            pltpu.VMEM((_W,), jnp.int32),
            pltpu.VMEM((_W,), jnp.int32),
            pltpu.VMEM((_W,), jnp.float32),
            pltpu.VMEM((_W,), jnp.float32),
            pltpu.VMEM((_W,), jnp.float32),
            pltpu.VMEM((_W,), jnp.float32),
            pltpu.VMEM((_W,), jnp.float32),
            pltpu.VMEM((_W,), jnp.float32),
            pltpu.VMEM_SHARED((3 * n_pad,), jnp.float32),
            pltpu.VMEM_SHARED((3 * n_pad,), jnp.float32),
            pltpu.SemaphoreType.DMA,
            pltpu.SemaphoreType.DMA,
            pltpu.SemaphoreType.DMA,
            pltpu.SemaphoreType.DMA,
        ],
    )
    def sc_kernel(dip_hbm, ii_hbm, jj_hbm, vx_hbm, vy_hbm, vz_hbm, out_hbm,
                  vecb, ib, jb, eib0, eib1, ejb0, ejb1, dib, djb,
                  cib0, cib1, cjb0, cjb1, acc, dipm,
                  sem_in, sem_g, sem_s0, sem_s1):
        c = lax.axis_index("c")
        s = lax.axis_index("s")
        wid = s * _NC + c
        zf = jnp.zeros((_L,), jnp.float32)
        eibs, ejbs = (eib0, eib1), (ejb0, ejb1)
        cibs, cjbs = (cib0, cib1), (cjb0, cjb1)
        sems = (sem_s0, sem_s1)
        n_my = (n_chunks - wid + _NW - 1) // _NW

        def zbuf_body(v, carry):
            cib0[pl.ds(v * _L, _L)] = zf
            return carry
        lax.fori_loop(0, _W // _L, zbuf_body, 0)
        w0_sub = s * words_per_sub

        def zacc_body(p, carry):
            pltpu.sync_copy(cib0, acc.at[pl.ds(w0_sub + p * _W, _W)])
            return carry
        lax.fori_loop(0, n_wb, zacc_body, 0)

        def stage_body(p, carry):
            off = w0_sub + p * _W
            pltpu.sync_copy(dip_hbm.at[pl.ds(off, _W)], cib0)
            pltpu.sync_copy(cib0, dipm.at[pl.ds(off, _W)])
            return carry
        lax.fori_loop(0, n_wb, stage_body, 0)
        plsc.subcore_barrier()

        def scatter_descs(h, issue):
            ds_list = []
            for k in range(_NIDX):
                sl = pl.ds(k * _IDX, _IDX)
                fn = pltpu.async_copy if issue else pltpu.make_async_copy
                kw = {"add": True} if issue else {}
                ds_list.append(fn(
                    cibs[h].at[sl], acc.at[eibs[h].at[sl]], sems[h], **kw))
                ds_list.append(fn(
                    cjbs[h].at[sl], acc.at[ejbs[h].at[sl]], sems[h], **kw))
            return ds_list

        def do_chunk(t, h):
            ch = wid + t * _NW
            base = ch * _CHUNK
            cps = [
                pltpu.async_copy(
                    vx_hbm.at[pl.ds(base, _CHUNK)],
                    vecb.at[pl.ds(0, _CHUNK)], sem_in),
                pltpu.async_copy(
                    vy_hbm.at[pl.ds(base, _CHUNK)],
                    vecb.at[pl.ds(_CHUNK, _CHUNK)], sem_in),
                pltpu.async_copy(
                    vz_hbm.at[pl.ds(base, _CHUNK)],
                    vecb.at[pl.ds(2 * _CHUNK, _CHUNK)], sem_in),
                pltpu.async_copy(
                    ii_hbm.at[pl.ds(base, _CHUNK)], ib, sem_in),
                pltpu.async_copy(
                    jj_hbm.at[pl.ds(base, _CHUNK)], jb, sem_in),
            ]
            for cp in cps:
                cp.wait()
            eib, ejb = eibs[h], ejbs[h]
            cib, cjb = cibs[h], cjbs[h]

            def exp_body(v, carry2):
                sl = pl.ds(v * _L, _L)
                iv = ib[sl]
                jv = jb[sl]
                eib[sl] = iv
                eib[pl.ds(_CHUNK + v * _L, _L)] = iv + n_pad
                eib[pl.ds(2 * _CHUNK + v * _L, _L)] = iv + 2 * n_pad
                ejb[sl] = jv
                ejb[pl.ds(_CHUNK + v * _L, _L)] = jv + n_pad
                ejb[pl.ds(2 * _CHUNK + v * _L, _L)] = jv + 2 * n_pad
                return carry2
            lax.fori_loop(0, _CHUNK // _L, exp_body, 0)

            gathers = []
            for k in range(_NIDX):
                sl = pl.ds(k * _IDX, _IDX)
                gathers.append(pltpu.async_copy(
                    dipm.at[eib.at[sl]], dib.at[sl], sem_g))
                gathers.append(pltpu.async_copy(
                    dipm.at[ejb.at[sl]], djb.at[sl], sem_g))
            for g in gathers:
                g.wait()

            def vec_body(v, carry2):
                s0 = pl.ds(v * _L, _L)
                s1 = pl.ds(_CHUNK + v * _L, _L)
                s2 = pl.ds(2 * _CHUNK + v * _L, _L)
                rx = vecb[s0]
                ry = vecb[s1]
                rz = vecb[s2]
                dix = dib[s0]
                diy = dib[s1]
                diz = dib[s2]
                djx = djb[s0]
                djy = djb[s1]
                djz = djb[s2]
                d2 = rx * rx + ry * ry + rz * rz
                y = _rsqrt(d2)
                y2 = y * y
                a3 = 0.5 * (y2 * y)
                a5 = 3.0 * (a3 * y2)
                sj = a5 * (rx * djx + ry * djy + rz * djz)
                cib[s0] = sj * rx - a3 * djx
                cib[s1] = sj * ry - a3 * djy
                cib[s2] = sj * rz - a3 * djz
                si = a5 * (rx * dix + ry * diy + rz * diz)
                cjb[s0] = si * rx - a3 * dix
                cjb[s1] = si * ry - a3 * diy
                cjb[s2] = si * rz - a3 * diz
                return carry2
            lax.fori_loop(0, _CHUNK // _L, vec_body, 0)

            scatter_descs(h, issue=True)

        def pair_body(t2, carry):
            for h in (0, 1):
                t = 2 * t2 + h

                @pl.when((t - 2 >= 0) & (t - 2 < n_my))
                def _drain():
                    for d in scatter_descs(h, issue=False):
                        d.wait()

                @pl.when(t < n_my)
                def _work():
                    do_chunk(t, h)
            return carry
        lax.fori_loop(0, n_pairs, pair_body, 0)

        for h in (0, 1):
            last_t = n_my - 1 - ((n_my - 1 - h) % 2)

            @pl.when((last_t >= 0) & (last_t + 2 >= (2 * n_pairs)))
            def _tail_drain():
                for d in scatter_descs(h, issue=False):
                    d.wait()
        plsc.subcore_barrier()

        def wb_body(p, carry):
            off = w0_sub + p * _W
            pltpu.sync_copy(acc.at[pl.ds(off, _W)], cib0)
            pltpu.sync_copy(cib0, out_hbm.at[c].at[pl.ds(off, _W)])
            return carry
        lax.fori_loop(0, n_wb, wb_body, 0)

    return sc_kernel(dipf, ii, jj, vx, vy, vz)


def _combine(partials):
    def body(i_ref, o_ref):
        o_ref[...] = i_ref[0] + i_ref[1]
    return pl.pallas_call(
        body,
        out_shape=jax.ShapeDtypeStruct(partials.shape[1:], jnp.float32),
    )(partials)


def kernel(dipoles, cell, positions, neighbor_indices, neighbor_vectors):
    n = dipoles.shape[0]
    e = neighbor_vectors.shape[0]
    assert e % _CHUNK == 0
    blk = _NS * _CHUNK
    n_pad = -(-n // blk) * blk
    idx_t = neighbor_indices.T
    ii = idx_t[0]
    jj = idx_t[1]
    vx = neighbor_vectors[:, 0]
    vy = neighbor_vectors[:, 1]
    vz = neighbor_vectors[:, 2]
    dip_t = dipoles.T
    dipf = jnp.zeros((3, n_pad), dipoles.dtype).at[:, :n].set(dip_t).reshape(-1)
    partials = _sc_edges(n, n_pad, e, dipf, ii, jj, vx, vy, vz)
    summed = _combine(partials.reshape(_NC, 3 * n_pad // 128, 128))
    return summed.reshape(3, n_pad)[:, :n].T
